# baseline (speedup 1.0000x reference)
import jax
import jax.numpy as jnp
from jax.experimental import pallas as pl
from jax.experimental.pallas import tpu as pltpu

MAX_LORAS = 8
MAX_RANK = 64
TM = 1024
NJ = 2


def _body(tok_ref, effc_ref, scalc_ref, x_ref, a_ref, b_ref, o_ref, h_bf):
    j = pl.program_id(1)
    R = MAX_LORAS * MAX_RANK
    dj = b_ref.shape[0] // NJ

    @pl.when(j == 0)
    def _stage1():
        xb = x_ref[...].astype(jnp.bfloat16)
        h = jax.lax.dot_general(
            xb, a_ref[...].astype(jnp.bfloat16), (((1,), (1,)), ((), ())),
            preferred_element_type=jnp.float32)
        col = jax.lax.broadcasted_iota(jnp.int32, (TM, R), 1)
        slot_of_col = jax.lax.shift_right_logical(col, 6)
        r_of_col = jnp.bitwise_and(col, MAX_RANK - 1)
        tok = tok_ref[0, 0, :]
        mask = ((slot_of_col == tok[:, None])
                & (r_of_col < effc_ref[0, :][None, :]))
        h = jnp.where(mask, h * scalc_ref[0, :][None, :], 0.0)
        h_bf[...] = h.astype(jnp.bfloat16)

    bj = b_ref[pl.ds(j * dj, dj), :].astype(jnp.bfloat16)
    o_ref[...] = jax.lax.dot_general(
        h_bf[...], bj, (((1,), (1,)), ((), ())),
        preferred_element_type=jnp.float32)


def kernel(x, lora_a, lora_b, lora_scaling, effective_rank, token_to_slot):
    T, d_in = x.shape
    E, r, _ = lora_a.shape
    d_out = lora_b.shape[1]
    R = E * r
    nblk = T // TM
    dj = d_out // NJ

    tok = token_to_slot.astype(jnp.int32).reshape(nblk, 1, TM)
    a_cat = lora_a.reshape(R, d_in)
    b_cat = lora_b.transpose(1, 0, 2).reshape(d_out, R)
    eff_cols = jnp.repeat(effective_rank, r).reshape(1, R)
    scal_cols = jnp.repeat(lora_scaling, r).reshape(1, R)

    out = pl.pallas_call(
        _body,
        grid=(nblk, NJ),
        in_specs=[
            pl.BlockSpec((1, 1, TM), lambda i, j: (i, 0, 0)),
            pl.BlockSpec((1, R), lambda i, j: (0, 0)),
            pl.BlockSpec((1, R), lambda i, j: (0, 0)),
            pl.BlockSpec((TM, d_in), lambda i, j: (i, 0)),
            pl.BlockSpec((R, d_in), lambda i, j: (0, 0)),
            pl.BlockSpec((d_out, R), lambda i, j: (0, 0)),
        ],
        out_specs=pl.BlockSpec((TM, dj), lambda i, j: (i, j)),
        out_shape=jax.ShapeDtypeStruct((T, d_out), jnp.float32),
        scratch_shapes=[pltpu.VMEM((TM, R), jnp.bfloat16)],
    )(tok, eff_cols, scal_cols, x, a_cat, b_cat)
    return out


# speedup vs baseline: 1.3501x; 1.3501x over previous
import jax
import jax.numpy as jnp
from jax.experimental import pallas as pl

MAX_LORAS = 8
MAX_RANK = 64
TM = 1024


def _body(tok_ref, effc_ref, scalc_ref, x_ref, a_ref, b_ref, o_ref):
    xb = x_ref[...].astype(jnp.bfloat16)
    h = jax.lax.dot_general(
        xb, a_ref[...].astype(jnp.bfloat16), (((1,), (1,)), ((), ())),
        preferred_element_type=jnp.float32)
    R = MAX_LORAS * MAX_RANK
    col = jax.lax.broadcasted_iota(jnp.int32, (TM, R), 1)
    slot_of_col = jax.lax.shift_right_logical(col, 6)
    r_of_col = jnp.bitwise_and(col, MAX_RANK - 1)
    tok = tok_ref[0, 0, :]
    mask = (slot_of_col == tok[:, None]) & (r_of_col < effc_ref[0, :][None, :])
    h = jnp.where(mask, h * scalc_ref[0, :][None, :], 0.0)
    o_ref[...] = jax.lax.dot_general(
        h.astype(jnp.bfloat16), b_ref[...].astype(jnp.bfloat16),
        (((1,), (1,)), ((), ())),
        preferred_element_type=jnp.float32)


def kernel(x, lora_a, lora_b, lora_scaling, effective_rank, token_to_slot):
    T, d_in = x.shape
    E, r, _ = lora_a.shape
    d_out = lora_b.shape[1]
    R = E * r
    nblk = T // TM

    tok = token_to_slot.astype(jnp.int32).reshape(nblk, 1, TM)
    a_cat = lora_a.reshape(R, d_in)
    b_cat = lora_b.transpose(1, 0, 2).reshape(d_out, R)
    eff_cols = jnp.repeat(effective_rank, r).reshape(1, R)
    scal_cols = jnp.repeat(lora_scaling, r).reshape(1, R)

    out = pl.pallas_call(
        _body,
        grid=(nblk,),
        in_specs=[
            pl.BlockSpec((1, 1, TM), lambda i: (i, 0, 0)),
            pl.BlockSpec((1, R), lambda i: (0, 0)),
            pl.BlockSpec((1, R), lambda i: (0, 0)),
            pl.BlockSpec((TM, d_in), lambda i: (i, 0)),
            pl.BlockSpec((R, d_in), lambda i: (0, 0)),
            pl.BlockSpec((d_out, R), lambda i: (0, 0)),
        ],
        out_specs=pl.BlockSpec((TM, d_out), lambda i: (i, 0)),
        out_shape=jax.ShapeDtypeStruct((T, d_out), jnp.float32),
    )(tok, eff_cols, scal_cols, x, a_cat, b_cat)
    return out
